# baseline (speedup 1.0000x reference)
import functools

import jax
import jax.numpy as jnp
from jax.experimental import pallas as pl

N_FACES = 100000
N_EDGES = 150000
D = 128


def _mm_body(a_ref, w_ref, o_ref, *, sigmoid_in, sigmoid_out):
    a = a_ref[...]
    if sigmoid_in:
        a = jax.nn.sigmoid(a)
    o = jnp.dot(a, w_ref[...], preferred_element_type=jnp.float32)
    if sigmoid_out:
        o = jax.nn.sigmoid(o)
    o_ref[...] = o


def _matmul(a, w, *, sigmoid_in=False, sigmoid_out=False, block=1000):
    n = a.shape[0]
    assert n % block == 0
    return pl.pallas_call(
        functools.partial(_mm_body, sigmoid_in=sigmoid_in, sigmoid_out=sigmoid_out),
        grid=(n // block,),
        in_specs=[
            pl.BlockSpec((block, D), lambda i: (i, 0)),
            pl.BlockSpec((D, D), lambda i: (0, 0)),
        ],
        out_specs=pl.BlockSpec((block, D), lambda i: (i, 0)),
        out_shape=jax.ShapeDtypeStruct((n, D), jnp.float32),
    )(a, w)


def _sigmoid_pallas(a, block=1000):
    n = a.shape[0]
    return pl.pallas_call(
        lambda a_ref, o_ref: o_ref.__setitem__(..., jax.nn.sigmoid(a_ref[...])),
        grid=(n // block,),
        in_specs=[pl.BlockSpec((block, D), lambda i: (i, 0))],
        out_specs=pl.BlockSpec((block, D), lambda i: (i, 0)),
        out_shape=jax.ShapeDtypeStruct((n, D), jnp.float32),
    )(a)


def kernel(x, rows, cols, vals, W1, W2):
    h = _matmul(x, W1)
    e = jnp.zeros((N_EDGES, D), jnp.float32).at[rows].add(
        jnp.take(h, cols, axis=0) * vals[:, None])
    h2 = _matmul(e, W2, sigmoid_in=True)
    out = jnp.zeros((N_FACES, D), jnp.float32).at[cols].add(
        jnp.take(h2, rows, axis=0) * vals[:, None])
    return _sigmoid_pallas(out)
